# baseline (speedup 1.0000x reference)
import numpy as np

import jax
import jax.numpy as jnp
from jax.experimental import pallas as pl
from jax.experimental.pallas import tpu as pltpu

_B, _C, _H, _W = 8, 192, 224, 224
_BS = 5
_HS, _WS = _H - (_BS - 1), _W - (_BS - 1)
_D = _B * _C
_COUNT_M = float(_D * _H * _W)
_SEEDS_PER_SAMPLE = _HS * _WS

_ROTS = ((13, 15, 26, 6), (17, 29, 16, 24))


def _threefry_key():
    def tf2x32(k0, k1, x0, x1):
        M = 0xFFFFFFFF
        ks = (k0, k1, 0x1BD11BDA ^ k0 ^ k1)
        x0 = (x0 + ks[0]) & M
        x1 = (x1 + ks[1]) & M
        for g in range(5):
            for r in _ROTS[g % 2]:
                x0 = (x0 + x1) & M
                x1 = ((x1 << r) | (x1 >> (32 - r))) & M
                x1 ^= x0
            x0 = (x0 + ks[(g + 1) % 3]) & M
            x1 = (x1 + ks[(g + 2) % 3] + g + 1) & M
        return x0, x1
    return tf2x32(0, 0, 0, 123)


_K0, _K1 = _threefry_key()
_K2 = 0x1BD11BDA ^ _K0 ^ _K1


def _random_bits(x1):
    ks = (np.uint32(_K0), np.uint32(_K1), np.uint32(_K2))
    x0 = jnp.full(x1.shape, ks[0], jnp.uint32)
    for g in range(5):
        for r in _ROTS[g % 2]:
            x0 = x0 + x1
            x1 = (x1 << np.uint32(r)) | (x1 >> np.uint32(32 - r))
            x1 = x1 ^ x0
        x0 = x0 + ks[(g + 1) % 3]
        x1 = x1 + np.uint32((int(ks[(g + 2) % 3]) + g + 1) & 0xFFFFFFFF)
    return x0 ^ x1


def _mask_kernel(lin_ref, ts_ref, m_ref, n_ref, mask_ref, cnt_ref, seed_ref):
    i = pl.program_id(0)

    @pl.when(i < _D)
    def _gen():
        base = (i.astype(jnp.uint32) * np.uint32(_SEEDS_PER_SAMPLE)
                + np.uint32(_K1))
        bits = _random_bits(lin_ref[...] + base)
        seed_ref[i & 1] = jnp.where(bits < ts_ref[...], jnp.float32(1.0),
                                    jnp.float32(0.0))

    @pl.when(i > 0)
    def _dilate():
        seed = seed_ref[(i & 1) ^ 1]
        colcnt = jnp.dot(seed, m_ref[...], preferred_element_type=jnp.float32)
        wincnt = jnp.dot(n_ref[...], colcnt, preferred_element_type=jnp.float32)
        keep = jnp.where(wincnt < jnp.float32(0.5), jnp.float32(1.0),
                         jnp.float32(0.0))
        mask_ref[0] = keep.astype(jnp.int8)
        tile_ones = jnp.sum(keep).astype(jnp.int32)

        @pl.when(i == 1)
        def _init():
            cnt_ref[0, 0] = tile_ones

        @pl.when(i > 1)
        def _acc():
            cnt_ref[0, 0] = cnt_ref[0, 0] + tile_ones


_APPLY_BLK = 8


def _apply_kernel(cnt_ref, x_ref, mask_ref, o_ref):
    scale = jnp.float32(_COUNT_M) / cnt_ref[0, 0].astype(jnp.float32)
    o_ref[...] = x_ref[...] * (mask_ref[...].astype(jnp.float32) * scale)


def kernel(x, gamma):
    xr = x.reshape(_D, _H, _W)

    ly = np.minimum(np.arange(_H), _HS - 1).astype(np.uint32)[:, None]
    lx = np.arange(_W, dtype=np.uint32)[None, :]
    lin = jnp.asarray(ly * np.uint32(_WS) + lx)
    thresh = (jnp.ceil(jnp.asarray(gamma, jnp.float32) * jnp.float32(8388608.0))
              .astype(jnp.uint32) << np.uint32(9))
    interior = jnp.asarray(
        ((np.arange(_H) < _HS)[:, None] & (np.arange(_W) < _WS)[None, :]))
    ts = jnp.where(interior, thresh, jnp.uint32(0))
    kk = np.arange(_H)
    m_mat = jnp.asarray(((kk[None, :] - kk[:, None] >= 0)
                         & (kk[None, :] - kk[:, None] <= _BS - 1))
                        .astype(np.float32))
    n_mat = m_mat.T

    mask, cnt = pl.pallas_call(
        _mask_kernel,
        grid=(_D + 1,),
        in_specs=[
            pl.BlockSpec((_H, _W), lambda i: (0, 0)),
            pl.BlockSpec((_H, _W), lambda i: (0, 0)),
            pl.BlockSpec((_H, _W), lambda i: (0, 0)),
            pl.BlockSpec((_H, _W), lambda i: (0, 0)),
        ],
        out_specs=[
            pl.BlockSpec((1, _H, _W), lambda i: (jnp.maximum(i - 1, 0), 0, 0)),
            pl.BlockSpec(memory_space=pltpu.SMEM),
        ],
        out_shape=[
            jax.ShapeDtypeStruct((_D, _H, _W), jnp.int8),
            jax.ShapeDtypeStruct((1, 1), jnp.int32),
        ],
        scratch_shapes=[pltpu.VMEM((2, _H, _W), jnp.float32)],
    )(lin, ts, m_mat, n_mat)

    out = pl.pallas_call(
        _apply_kernel,
        grid=(_D // _APPLY_BLK,),
        in_specs=[
            pl.BlockSpec(memory_space=pltpu.SMEM),
            pl.BlockSpec((_APPLY_BLK, _H, _W), lambda i: (i, 0, 0)),
            pl.BlockSpec((_APPLY_BLK, _H, _W), lambda i: (i, 0, 0)),
        ],
        out_specs=pl.BlockSpec((_APPLY_BLK, _H, _W), lambda i: (i, 0, 0)),
        out_shape=jax.ShapeDtypeStruct((_D, _H, _W), jnp.float32),
    )(cnt, xr, mask)

    return out.reshape(x.shape)


# speedup vs baseline: 1.4412x; 1.4412x over previous
import numpy as np

import jax
import jax.numpy as jnp
from jax.experimental import pallas as pl
from jax.experimental.pallas import tpu as pltpu

_B, _C, _H, _W = 8, 192, 224, 224
_BS = 5
_HS, _WS = _H - (_BS - 1), _W - (_BS - 1)
_D = _B * _C
_COUNT_M = float(_D * _H * _W)
_SEEDS_PER_SAMPLE = _HS * _WS

_ROTS = ((13, 15, 26, 6), (17, 29, 16, 24))


def _threefry_key():
    def tf2x32(k0, k1, x0, x1):
        M = 0xFFFFFFFF
        ks = (k0, k1, 0x1BD11BDA ^ k0 ^ k1)
        x0 = (x0 + ks[0]) & M
        x1 = (x1 + ks[1]) & M
        for g in range(5):
            for r in _ROTS[g % 2]:
                x0 = (x0 + x1) & M
                x1 = ((x1 << r) | (x1 >> (32 - r))) & M
                x1 ^= x0
            x0 = (x0 + ks[(g + 1) % 3]) & M
            x1 = (x1 + ks[(g + 2) % 3] + g + 1) & M
        return x0, x1
    return tf2x32(0, 0, 0, 123)


_K0, _K1 = _threefry_key()
_K2 = 0x1BD11BDA ^ _K0 ^ _K1


def _random_bits(x1):
    ks = (np.uint32(_K0), np.uint32(_K1), np.uint32(_K2))
    x0 = jnp.full(x1.shape, ks[0], jnp.uint32)
    for g in range(5):
        for r in _ROTS[g % 2]:
            x0 = x0 + x1
            x1 = (x1 << np.uint32(r)) | (x1 >> np.uint32(32 - r))
            x1 = x1 ^ x0
        x0 = x0 + ks[(g + 1) % 3]
        x1 = x1 + np.uint32((int(ks[(g + 2) % 3]) + g + 1) & 0xFFFFFFFF)
    return x0 ^ x1


_GEN_BLK = 8


def _mask_kernel(lin_ref, ts_ref, m_ref, n_ref, mask_ref, cnt_ref):
    i = pl.program_id(0)
    total = jnp.int32(0)
    for k in range(_GEN_BLK):
        s_idx = i * _GEN_BLK + k
        base = (s_idx.astype(jnp.uint32) * np.uint32(_SEEDS_PER_SAMPLE)
                + np.uint32(_K1))
        bits = _random_bits(lin_ref[...] + base)
        seed = jnp.where(bits < ts_ref[...], jnp.float32(1.0),
                         jnp.float32(0.0))
        colcnt = jnp.dot(seed, m_ref[...], preferred_element_type=jnp.float32)
        wincnt = jnp.dot(n_ref[...], colcnt, preferred_element_type=jnp.float32)
        keep = jnp.where(wincnt < jnp.float32(0.5), jnp.float32(1.0),
                         jnp.float32(0.0))
        mask_ref[k] = keep.astype(jnp.int8)
        total = total + jnp.sum(keep).astype(jnp.int32)
    prev = jnp.where(i == 0, jnp.int32(0), cnt_ref[0, 0])
    cnt_ref[0, 0] = prev + total


_APPLY_BLK = 8


def _apply_kernel(cnt_ref, x_ref, mask_ref, o_ref):
    scale = jnp.float32(_COUNT_M) / cnt_ref[0, 0].astype(jnp.float32)
    o_ref[...] = x_ref[...] * (mask_ref[...].astype(jnp.float32) * scale)


def kernel(x, gamma):
    xr = x.reshape(_D, _H, _W)

    ly = np.minimum(np.arange(_H), _HS - 1).astype(np.uint32)[:, None]
    lx = np.arange(_W, dtype=np.uint32)[None, :]
    lin = jnp.asarray(ly * np.uint32(_WS) + lx)
    thresh = (jnp.ceil(jnp.asarray(gamma, jnp.float32) * jnp.float32(8388608.0))
              .astype(jnp.uint32) << np.uint32(9))
    interior = jnp.asarray(
        ((np.arange(_H) < _HS)[:, None] & (np.arange(_W) < _WS)[None, :]))
    ts = jnp.where(interior, thresh, jnp.uint32(0))
    kk = np.arange(_H)
    m_mat = jnp.asarray(((kk[None, :] - kk[:, None] >= 0)
                         & (kk[None, :] - kk[:, None] <= _BS - 1))
                        .astype(np.float32))
    n_mat = m_mat.T

    mask, cnt = pl.pallas_call(
        _mask_kernel,
        grid=(_D // _GEN_BLK,),
        in_specs=[
            pl.BlockSpec((_H, _W), lambda i: (0, 0)),
            pl.BlockSpec((_H, _W), lambda i: (0, 0)),
            pl.BlockSpec((_H, _W), lambda i: (0, 0)),
            pl.BlockSpec((_H, _W), lambda i: (0, 0)),
        ],
        out_specs=[
            pl.BlockSpec((_GEN_BLK, _H, _W), lambda i: (i, 0, 0)),
            pl.BlockSpec(memory_space=pltpu.SMEM),
        ],
        out_shape=[
            jax.ShapeDtypeStruct((_D, _H, _W), jnp.int8),
            jax.ShapeDtypeStruct((1, 1), jnp.int32),
        ],
    )(lin, ts, m_mat, n_mat)

    out = pl.pallas_call(
        _apply_kernel,
        grid=(_D // _APPLY_BLK,),
        in_specs=[
            pl.BlockSpec(memory_space=pltpu.SMEM),
            pl.BlockSpec((_APPLY_BLK, _H, _W), lambda i: (i, 0, 0)),
            pl.BlockSpec((_APPLY_BLK, _H, _W), lambda i: (i, 0, 0)),
        ],
        out_specs=pl.BlockSpec((_APPLY_BLK, _H, _W), lambda i: (i, 0, 0)),
        out_shape=jax.ShapeDtypeStruct((_D, _H, _W), jnp.float32),
    )(cnt, xr, mask)

    return out.reshape(x.shape)


# speedup vs baseline: 1.4494x; 1.0057x over previous
import numpy as np

import jax
import jax.numpy as jnp
from jax.experimental import pallas as pl
from jax.experimental.pallas import tpu as pltpu

_B, _C, _H, _W = 8, 192, 224, 224
_BS = 5
_HS, _WS = _H - (_BS - 1), _W - (_BS - 1)
_D = _B * _C
_COUNT_M = float(_D * _H * _W)
_SEEDS_PER_SAMPLE = _HS * _WS

_ROTS = ((13, 15, 26, 6), (17, 29, 16, 24))


def _threefry_key():
    def tf2x32(k0, k1, x0, x1):
        M = 0xFFFFFFFF
        ks = (k0, k1, 0x1BD11BDA ^ k0 ^ k1)
        x0 = (x0 + ks[0]) & M
        x1 = (x1 + ks[1]) & M
        for g in range(5):
            for r in _ROTS[g % 2]:
                x0 = (x0 + x1) & M
                x1 = ((x1 << r) | (x1 >> (32 - r))) & M
                x1 ^= x0
            x0 = (x0 + ks[(g + 1) % 3]) & M
            x1 = (x1 + ks[(g + 2) % 3] + g + 1) & M
        return x0, x1
    return tf2x32(0, 0, 0, 123)


_K0, _K1 = _threefry_key()
_K2 = 0x1BD11BDA ^ _K0 ^ _K1


def _random_bits(x1):
    ks = (np.uint32(_K0), np.uint32(_K1), np.uint32(_K2))
    x0 = jnp.full(x1.shape, ks[0], jnp.uint32)
    for g in range(5):
        for r in _ROTS[g % 2]:
            x0 = x0 + x1
            x1 = (x1 << np.uint32(r)) | (x1 >> np.uint32(32 - r))
            x1 = x1 ^ x0
        x0 = x0 + ks[(g + 1) % 3]
        x1 = x1 + np.uint32((int(ks[(g + 2) % 3]) + g + 1) & 0xFFFFFFFF)
    return x0 ^ x1


_GEN_BLK = 8


def _mask_kernel(lin_ref, ts_ref, m_ref, n_ref, mask_ref, cnt_ref):
    i = pl.program_id(0)
    packed = jnp.zeros((_H, _W), jnp.float32)
    ones_acc = jnp.zeros((_H, _W), jnp.float32)
    for k in range(_GEN_BLK):
        s_idx = i * _GEN_BLK + k
        base = (s_idx.astype(jnp.uint32) * np.uint32(_SEEDS_PER_SAMPLE)
                + np.uint32(_K1))
        bits = _random_bits(lin_ref[...] + base)
        seed = jnp.where(bits < ts_ref[...], jnp.float32(1.0),
                         jnp.float32(0.0))
        colcnt = jnp.dot(seed, m_ref[...], preferred_element_type=jnp.float32)
        wincnt = jnp.dot(n_ref[...], colcnt, preferred_element_type=jnp.float32)
        keep = jnp.where(wincnt < jnp.float32(0.5), jnp.float32(1.0),
                         jnp.float32(0.0))
        packed = packed + keep * jnp.float32(1 << k)
        ones_acc = ones_acc + keep
    mask_ref[0] = packed.astype(jnp.int32)
    tile_ones = jnp.sum(ones_acc).astype(jnp.int32)
    prev = jnp.where(i == 0, jnp.int32(0), cnt_ref[0, 0])
    cnt_ref[0, 0] = prev + tile_ones


_APPLY_BLK = 8


def _apply_kernel(cnt_ref, x_ref, mask_ref, o_ref):
    scale = jnp.float32(_COUNT_M) / cnt_ref[0, 0].astype(jnp.float32)
    packed = mask_ref[0]
    for k in range(_APPLY_BLK):
        bit = packed & jnp.int32(1 << k)
        xs = x_ref[k] * scale
        o_ref[k] = jnp.where(bit != 0, xs, jnp.float32(0.0))


def kernel(x, gamma):
    xr = x.reshape(_D, _H, _W)

    ly = np.minimum(np.arange(_H), _HS - 1).astype(np.uint32)[:, None]
    lx = np.arange(_W, dtype=np.uint32)[None, :]
    lin = jnp.asarray(ly * np.uint32(_WS) + lx)
    thresh = (jnp.ceil(jnp.asarray(gamma, jnp.float32) * jnp.float32(8388608.0))
              .astype(jnp.uint32) << np.uint32(9))
    interior = jnp.asarray(
        ((np.arange(_H) < _HS)[:, None] & (np.arange(_W) < _WS)[None, :]))
    ts = jnp.where(interior, thresh, jnp.uint32(0))
    kk = np.arange(_H)
    m_mat = jnp.asarray(((kk[None, :] - kk[:, None] >= 0)
                         & (kk[None, :] - kk[:, None] <= _BS - 1))
                        .astype(np.float32))
    n_mat = m_mat.T

    mask, cnt = pl.pallas_call(
        _mask_kernel,
        grid=(_D // _GEN_BLK,),
        in_specs=[
            pl.BlockSpec((_H, _W), lambda i: (0, 0)),
            pl.BlockSpec((_H, _W), lambda i: (0, 0)),
            pl.BlockSpec((_H, _W), lambda i: (0, 0)),
            pl.BlockSpec((_H, _W), lambda i: (0, 0)),
        ],
        out_specs=[
            pl.BlockSpec((1, _H, _W), lambda i: (i, 0, 0)),
            pl.BlockSpec(memory_space=pltpu.SMEM),
        ],
        out_shape=[
            jax.ShapeDtypeStruct((_D // _GEN_BLK, _H, _W), jnp.int32),
            jax.ShapeDtypeStruct((1, 1), jnp.int32),
        ],
    )(lin, ts, m_mat, n_mat)

    out = pl.pallas_call(
        _apply_kernel,
        grid=(_D // _APPLY_BLK,),
        in_specs=[
            pl.BlockSpec(memory_space=pltpu.SMEM),
            pl.BlockSpec((_APPLY_BLK, _H, _W), lambda i: (i, 0, 0)),
            pl.BlockSpec((1, _H, _W), lambda i: (i, 0, 0)),
        ],
        out_specs=pl.BlockSpec((_APPLY_BLK, _H, _W), lambda i: (i, 0, 0)),
        out_shape=jax.ShapeDtypeStruct((_D, _H, _W), jnp.float32),
    )(cnt, xr, mask)

    return out.reshape(x.shape)


# speedup vs baseline: 1.5111x; 1.0425x over previous
import numpy as np

import jax
import jax.numpy as jnp
from jax.experimental import pallas as pl
from jax.experimental.pallas import tpu as pltpu

_B, _C, _H, _W = 8, 192, 224, 224
_BS = 5
_HS, _WS = _H - (_BS - 1), _W - (_BS - 1)
_D = _B * _C
_COUNT_M = float(_D * _H * _W)
_SEEDS_PER_SAMPLE = _HS * _WS

_ROTS = ((13, 15, 26, 6), (17, 29, 16, 24))


def _threefry_key():
    def tf2x32(k0, k1, x0, x1):
        M = 0xFFFFFFFF
        ks = (k0, k1, 0x1BD11BDA ^ k0 ^ k1)
        x0 = (x0 + ks[0]) & M
        x1 = (x1 + ks[1]) & M
        for g in range(5):
            for r in _ROTS[g % 2]:
                x0 = (x0 + x1) & M
                x1 = ((x1 << r) | (x1 >> (32 - r))) & M
                x1 ^= x0
            x0 = (x0 + ks[(g + 1) % 3]) & M
            x1 = (x1 + ks[(g + 2) % 3] + g + 1) & M
        return x0, x1
    return tf2x32(0, 0, 0, 123)


_K0, _K1 = _threefry_key()
_K2 = 0x1BD11BDA ^ _K0 ^ _K1


def _random_bits(x1):
    ks = (np.uint32(_K0), np.uint32(_K1), np.uint32(_K2))
    x0 = jnp.full(x1.shape, ks[0], jnp.uint32)
    for g in range(5):
        for r in _ROTS[g % 2]:
            x0 = x0 + x1
            x1 = (x1 << np.uint32(r)) | (x1 >> np.uint32(32 - r))
            x1 = x1 ^ x0
        x0 = x0 + ks[(g + 1) % 3]
        x1 = x1 + np.uint32((int(ks[(g + 2) % 3]) + g + 1) & 0xFFFFFFFF)
    return x0 ^ x1


_GEN_BLK = 16


def _mask_kernel(lin_ref, ts_ref, m_ref, n_ref, mask_ref, cnt_ref):
    i = pl.program_id(0)
    packed = jnp.zeros((_H, _W), jnp.float32)
    ones_acc = jnp.zeros((_H, _W), jnp.float32)
    for k in range(_GEN_BLK):
        s_idx = i * _GEN_BLK + k
        base = (s_idx.astype(jnp.uint32) * np.uint32(_SEEDS_PER_SAMPLE)
                + np.uint32(_K1))
        bits = _random_bits(lin_ref[...] + base)
        seed = jnp.where(bits < ts_ref[...], jnp.float32(1.0),
                         jnp.float32(0.0))
        colcnt = jnp.dot(seed, m_ref[...], preferred_element_type=jnp.float32)
        wincnt = jnp.dot(n_ref[...], colcnt, preferred_element_type=jnp.float32)
        keep = jnp.where(wincnt < jnp.float32(0.5), jnp.float32(1.0),
                         jnp.float32(0.0))
        packed = packed + keep * jnp.float32(1 << k)
        ones_acc = ones_acc + keep
    mask_ref[0] = packed.astype(jnp.int32)
    tile_ones = jnp.sum(ones_acc).astype(jnp.int32)
    prev = jnp.where(i == 0, jnp.int32(0), cnt_ref[0, 0])
    cnt_ref[0, 0] = prev + tile_ones


_APPLY_BLK = 16


def _apply_kernel(cnt_ref, x_ref, mask_ref, o_ref):
    scale = jnp.float32(_COUNT_M) / cnt_ref[0, 0].astype(jnp.float32)
    packed = mask_ref[0]
    for k in range(_APPLY_BLK):
        bit = packed & jnp.int32(1 << k)
        xs = x_ref[k] * scale
        o_ref[k] = jnp.where(bit != 0, xs, jnp.float32(0.0))


def kernel(x, gamma):
    xr = x.reshape(_D, _H, _W)

    ly = np.minimum(np.arange(_H), _HS - 1).astype(np.uint32)[:, None]
    lx = np.arange(_W, dtype=np.uint32)[None, :]
    lin = jnp.asarray(ly * np.uint32(_WS) + lx)
    thresh = (jnp.ceil(jnp.asarray(gamma, jnp.float32) * jnp.float32(8388608.0))
              .astype(jnp.uint32) << np.uint32(9))
    interior = jnp.asarray(
        ((np.arange(_H) < _HS)[:, None] & (np.arange(_W) < _WS)[None, :]))
    ts = jnp.where(interior, thresh, jnp.uint32(0))
    kk = np.arange(_H)
    m_mat = jnp.asarray(((kk[None, :] - kk[:, None] >= 0)
                         & (kk[None, :] - kk[:, None] <= _BS - 1))
                        .astype(np.float32))
    n_mat = m_mat.T

    mask, cnt = pl.pallas_call(
        _mask_kernel,
        grid=(_D // _GEN_BLK,),
        in_specs=[
            pl.BlockSpec((_H, _W), lambda i: (0, 0)),
            pl.BlockSpec((_H, _W), lambda i: (0, 0)),
            pl.BlockSpec((_H, _W), lambda i: (0, 0)),
            pl.BlockSpec((_H, _W), lambda i: (0, 0)),
        ],
        out_specs=[
            pl.BlockSpec((1, _H, _W), lambda i: (i, 0, 0)),
            pl.BlockSpec(memory_space=pltpu.SMEM),
        ],
        out_shape=[
            jax.ShapeDtypeStruct((_D // _GEN_BLK, _H, _W), jnp.int32),
            jax.ShapeDtypeStruct((1, 1), jnp.int32),
        ],
    )(lin, ts, m_mat, n_mat)

    out = pl.pallas_call(
        _apply_kernel,
        grid=(_D // _APPLY_BLK,),
        in_specs=[
            pl.BlockSpec(memory_space=pltpu.SMEM),
            pl.BlockSpec((_APPLY_BLK, _H, _W), lambda i: (i, 0, 0)),
            pl.BlockSpec((1, _H, _W), lambda i: (i, 0, 0)),
        ],
        out_specs=pl.BlockSpec((_APPLY_BLK, _H, _W), lambda i: (i, 0, 0)),
        out_shape=jax.ShapeDtypeStruct((_D, _H, _W), jnp.float32),
    )(cnt, xr, mask)

    return out.reshape(x.shape)


# speedup vs baseline: 1.5209x; 1.0065x over previous
import numpy as np

import jax
import jax.numpy as jnp
from jax.experimental import pallas as pl
from jax.experimental.pallas import tpu as pltpu

_B, _C, _H, _W = 8, 192, 224, 224
_BS = 5
_HS, _WS = _H - (_BS - 1), _W - (_BS - 1)
_D = _B * _C
_COUNT_M = float(_D * _H * _W)
_SEEDS_PER_SAMPLE = _HS * _WS

_ROTS = ((13, 15, 26, 6), (17, 29, 16, 24))


def _threefry_key():
    def tf2x32(k0, k1, x0, x1):
        M = 0xFFFFFFFF
        ks = (k0, k1, 0x1BD11BDA ^ k0 ^ k1)
        x0 = (x0 + ks[0]) & M
        x1 = (x1 + ks[1]) & M
        for g in range(5):
            for r in _ROTS[g % 2]:
                x0 = (x0 + x1) & M
                x1 = ((x1 << r) | (x1 >> (32 - r))) & M
                x1 ^= x0
            x0 = (x0 + ks[(g + 1) % 3]) & M
            x1 = (x1 + ks[(g + 2) % 3] + g + 1) & M
        return x0, x1
    return tf2x32(0, 0, 0, 123)


_K0, _K1 = _threefry_key()
_K2 = 0x1BD11BDA ^ _K0 ^ _K1


def _random_bits(x1):
    ks = (np.uint32(_K0), np.uint32(_K1), np.uint32(_K2))
    x0 = jnp.full(x1.shape, ks[0], jnp.uint32)
    for g in range(5):
        for r in _ROTS[g % 2]:
            x0 = x0 + x1
            x1 = (x1 << np.uint32(r)) | (x1 >> np.uint32(32 - r))
            x1 = x1 ^ x0
        x0 = x0 + ks[(g + 1) % 3]
        x1 = x1 + np.uint32((int(ks[(g + 2) % 3]) + g + 1) & 0xFFFFFFFF)
    return x0 ^ x1


_GEN_BLK = 16


def _mask_kernel(lin_ref, ts_ref, m_ref, n_ref, mask_ref, cnt_ref):
    i = pl.program_id(0)
    packed = jnp.zeros((_H, _W), jnp.float32)
    ones_acc = jnp.zeros((_H, _W), jnp.float32)
    for k in range(_GEN_BLK):
        s_idx = i * _GEN_BLK + k
        base = (s_idx.astype(jnp.uint32) * np.uint32(_SEEDS_PER_SAMPLE)
                + np.uint32(_K1))
        bits = _random_bits(lin_ref[...] + base)
        seed = jnp.where(bits < ts_ref[...], jnp.float32(1.0),
                         jnp.float32(0.0))
        colcnt = jnp.dot(seed, m_ref[...], preferred_element_type=jnp.float32)
        wincnt = jnp.dot(n_ref[...], colcnt, preferred_element_type=jnp.float32)
        keep = jnp.where(wincnt < jnp.float32(0.5), jnp.float32(1.0),
                         jnp.float32(0.0))
        packed = packed + keep * jnp.float32(1 << k)
        ones_acc = ones_acc + keep
    mask_ref[0] = packed.astype(jnp.int32)
    tile_ones = jnp.sum(ones_acc).astype(jnp.int32)
    prev = jnp.where(i == 0, jnp.int32(0), cnt_ref[0, 0])
    cnt_ref[0, 0] = prev + tile_ones


_APPLY_BLK = 32


def _apply_kernel(cnt_ref, x_ref, mask_ref, o_ref):
    scale = jnp.float32(_COUNT_M) / cnt_ref[0, 0].astype(jnp.float32)
    for p in range(_APPLY_BLK // _GEN_BLK):
        packed = mask_ref[p]
        for k in range(_GEN_BLK):
            s = p * _GEN_BLK + k
            bit = packed & jnp.int32(1 << k)
            xs = x_ref[s] * scale
            o_ref[s] = jnp.where(bit != 0, xs, jnp.float32(0.0))


def kernel(x, gamma):
    xr = x.reshape(_D, _H, _W)

    ly = np.minimum(np.arange(_H), _HS - 1).astype(np.uint32)[:, None]
    lx = np.arange(_W, dtype=np.uint32)[None, :]
    lin = jnp.asarray(ly * np.uint32(_WS) + lx)
    thresh = (jnp.ceil(jnp.asarray(gamma, jnp.float32) * jnp.float32(8388608.0))
              .astype(jnp.uint32) << np.uint32(9))
    interior = jnp.asarray(
        ((np.arange(_H) < _HS)[:, None] & (np.arange(_W) < _WS)[None, :]))
    ts = jnp.where(interior, thresh, jnp.uint32(0))
    kk = np.arange(_H)
    m_mat = jnp.asarray(((kk[None, :] - kk[:, None] >= 0)
                         & (kk[None, :] - kk[:, None] <= _BS - 1))
                        .astype(np.float32))
    n_mat = m_mat.T

    mask, cnt = pl.pallas_call(
        _mask_kernel,
        grid=(_D // _GEN_BLK,),
        in_specs=[
            pl.BlockSpec((_H, _W), lambda i: (0, 0)),
            pl.BlockSpec((_H, _W), lambda i: (0, 0)),
            pl.BlockSpec((_H, _W), lambda i: (0, 0)),
            pl.BlockSpec((_H, _W), lambda i: (0, 0)),
        ],
        out_specs=[
            pl.BlockSpec((1, _H, _W), lambda i: (i, 0, 0)),
            pl.BlockSpec(memory_space=pltpu.SMEM),
        ],
        out_shape=[
            jax.ShapeDtypeStruct((_D // _GEN_BLK, _H, _W), jnp.int32),
            jax.ShapeDtypeStruct((1, 1), jnp.int32),
        ],
    )(lin, ts, m_mat, n_mat)

    out = pl.pallas_call(
        _apply_kernel,
        grid=(_D // _APPLY_BLK,),
        in_specs=[
            pl.BlockSpec(memory_space=pltpu.SMEM),
            pl.BlockSpec((_APPLY_BLK, _H, _W), lambda i: (i, 0, 0)),
            pl.BlockSpec((_APPLY_BLK // _GEN_BLK, _H, _W), lambda i: (i, 0, 0)),
        ],
        out_specs=pl.BlockSpec((_APPLY_BLK, _H, _W), lambda i: (i, 0, 0)),
        out_shape=jax.ShapeDtypeStruct((_D, _H, _W), jnp.float32),
    )(cnt, xr, mask)

    return out.reshape(x.shape)


# speedup vs baseline: 1.5282x; 1.0048x over previous
import numpy as np

import jax
import jax.numpy as jnp
from jax.experimental import pallas as pl
from jax.experimental.pallas import tpu as pltpu

_B, _C, _H, _W = 8, 192, 224, 224
_BS = 5
_HS, _WS = _H - (_BS - 1), _W - (_BS - 1)
_D = _B * _C
_COUNT_M = float(_D * _H * _W)
_SEEDS_PER_SAMPLE = _HS * _WS

_ROTS = ((13, 15, 26, 6), (17, 29, 16, 24))


def _threefry_key():
    def tf2x32(k0, k1, x0, x1):
        M = 0xFFFFFFFF
        ks = (k0, k1, 0x1BD11BDA ^ k0 ^ k1)
        x0 = (x0 + ks[0]) & M
        x1 = (x1 + ks[1]) & M
        for g in range(5):
            for r in _ROTS[g % 2]:
                x0 = (x0 + x1) & M
                x1 = ((x1 << r) | (x1 >> (32 - r))) & M
                x1 ^= x0
            x0 = (x0 + ks[(g + 1) % 3]) & M
            x1 = (x1 + ks[(g + 2) % 3] + g + 1) & M
        return x0, x1
    return tf2x32(0, 0, 0, 123)


_K0, _K1 = _threefry_key()
_K2 = 0x1BD11BDA ^ _K0 ^ _K1


def _random_bits(x1):
    ks = (np.uint32(_K0), np.uint32(_K1), np.uint32(_K2))
    x0 = jnp.full(x1.shape, ks[0], jnp.uint32)
    for g in range(5):
        for r in _ROTS[g % 2]:
            x0 = x0 + x1
            x1 = (x1 << np.uint32(r)) | (x1 >> np.uint32(32 - r))
            x1 = x1 ^ x0
        x0 = x0 + ks[(g + 1) % 3]
        x1 = x1 + np.uint32((int(ks[(g + 2) % 3]) + g + 1) & 0xFFFFFFFF)
    return x0 ^ x1


_GEN_BLK = 24


def _mask_kernel(lin_ref, ts_ref, m_ref, n_ref, mask_ref, cnt_ref):
    i = pl.program_id(0)
    packed = jnp.zeros((_H, _W), jnp.float32)
    ones_acc = jnp.zeros((_H, _W), jnp.float32)
    for k in range(_GEN_BLK):
        s_idx = i * _GEN_BLK + k
        base = (s_idx.astype(jnp.uint32) * np.uint32(_SEEDS_PER_SAMPLE)
                + np.uint32(_K1))
        bits = _random_bits(lin_ref[...] + base)
        seed = jnp.where(bits < ts_ref[...], jnp.float32(1.0),
                         jnp.float32(0.0))
        colcnt = jnp.dot(seed, m_ref[...], preferred_element_type=jnp.float32)
        wincnt = jnp.dot(n_ref[...], colcnt, preferred_element_type=jnp.float32)
        keep = jnp.where(wincnt < jnp.float32(0.5), jnp.float32(1.0),
                         jnp.float32(0.0))
        packed = packed + keep * jnp.float32(1 << k)
        ones_acc = ones_acc + keep
    mask_ref[0] = packed.astype(jnp.int32)
    tile_ones = jnp.sum(ones_acc).astype(jnp.int32)
    prev = jnp.where(i == 0, jnp.int32(0), cnt_ref[0, 0])
    cnt_ref[0, 0] = prev + tile_ones


_APPLY_BLK = 24


def _apply_kernel(cnt_ref, x_ref, mask_ref, o_ref):
    scale = jnp.float32(_COUNT_M) / cnt_ref[0, 0].astype(jnp.float32)
    for p in range(_APPLY_BLK // _GEN_BLK):
        packed = mask_ref[p]
        for k in range(_GEN_BLK):
            s = p * _GEN_BLK + k
            bit = packed & jnp.int32(1 << k)
            xs = x_ref[s] * scale
            o_ref[s] = jnp.where(bit != 0, xs, jnp.float32(0.0))


def kernel(x, gamma):
    xr = x.reshape(_D, _H, _W)

    ly = np.minimum(np.arange(_H), _HS - 1).astype(np.uint32)[:, None]
    lx = np.arange(_W, dtype=np.uint32)[None, :]
    lin = jnp.asarray(ly * np.uint32(_WS) + lx)
    thresh = (jnp.ceil(jnp.asarray(gamma, jnp.float32) * jnp.float32(8388608.0))
              .astype(jnp.uint32) << np.uint32(9))
    interior = jnp.asarray(
        ((np.arange(_H) < _HS)[:, None] & (np.arange(_W) < _WS)[None, :]))
    ts = jnp.where(interior, thresh, jnp.uint32(0))
    kk = np.arange(_H)
    m_mat = jnp.asarray(((kk[None, :] - kk[:, None] >= 0)
                         & (kk[None, :] - kk[:, None] <= _BS - 1))
                        .astype(np.float32))
    n_mat = m_mat.T

    mask, cnt = pl.pallas_call(
        _mask_kernel,
        grid=(_D // _GEN_BLK,),
        in_specs=[
            pl.BlockSpec((_H, _W), lambda i: (0, 0)),
            pl.BlockSpec((_H, _W), lambda i: (0, 0)),
            pl.BlockSpec((_H, _W), lambda i: (0, 0)),
            pl.BlockSpec((_H, _W), lambda i: (0, 0)),
        ],
        out_specs=[
            pl.BlockSpec((1, _H, _W), lambda i: (i, 0, 0)),
            pl.BlockSpec(memory_space=pltpu.SMEM),
        ],
        out_shape=[
            jax.ShapeDtypeStruct((_D // _GEN_BLK, _H, _W), jnp.int32),
            jax.ShapeDtypeStruct((1, 1), jnp.int32),
        ],
    )(lin, ts, m_mat, n_mat)

    out = pl.pallas_call(
        _apply_kernel,
        grid=(_D // _APPLY_BLK,),
        in_specs=[
            pl.BlockSpec(memory_space=pltpu.SMEM),
            pl.BlockSpec((_APPLY_BLK, _H, _W), lambda i: (i, 0, 0)),
            pl.BlockSpec((_APPLY_BLK // _GEN_BLK, _H, _W), lambda i: (i, 0, 0)),
        ],
        out_specs=pl.BlockSpec((_APPLY_BLK, _H, _W), lambda i: (i, 0, 0)),
        out_shape=jax.ShapeDtypeStruct((_D, _H, _W), jnp.float32),
    )(cnt, xr, mask)

    return out.reshape(x.shape)


# speedup vs baseline: 1.5296x; 1.0009x over previous
import numpy as np

import jax
import jax.numpy as jnp
from jax.experimental import pallas as pl
from jax.experimental.pallas import tpu as pltpu

_B, _C, _H, _W = 8, 192, 224, 224
_BS = 5
_HS, _WS = _H - (_BS - 1), _W - (_BS - 1)
_D = _B * _C
_COUNT_M = float(_D * _H * _W)
_SEEDS_PER_SAMPLE = _HS * _WS

_ROTS = ((13, 15, 26, 6), (17, 29, 16, 24))


def _threefry_key():
    def tf2x32(k0, k1, x0, x1):
        M = 0xFFFFFFFF
        ks = (k0, k1, 0x1BD11BDA ^ k0 ^ k1)
        x0 = (x0 + ks[0]) & M
        x1 = (x1 + ks[1]) & M
        for g in range(5):
            for r in _ROTS[g % 2]:
                x0 = (x0 + x1) & M
                x1 = ((x1 << r) | (x1 >> (32 - r))) & M
                x1 ^= x0
            x0 = (x0 + ks[(g + 1) % 3]) & M
            x1 = (x1 + ks[(g + 2) % 3] + g + 1) & M
        return x0, x1
    return tf2x32(0, 0, 0, 123)


_K0, _K1 = _threefry_key()
_K2 = 0x1BD11BDA ^ _K0 ^ _K1


def _random_bits(x1):
    ks = (np.uint32(_K0), np.uint32(_K1), np.uint32(_K2))
    x0 = jnp.full(x1.shape, ks[0], jnp.uint32)
    for g in range(5):
        for r in _ROTS[g % 2]:
            x0 = x0 + x1
            x1 = (x1 << np.uint32(r)) | (x1 >> np.uint32(32 - r))
            x1 = x1 ^ x0
        x0 = x0 + ks[(g + 1) % 3]
        x1 = x1 + np.uint32((int(ks[(g + 2) % 3]) + g + 1) & 0xFFFFFFFF)
    return x0 ^ x1


_GEN_BLK = 24


def _mask_kernel(lin_ref, ts_ref, m_ref, n_ref, mask_ref, cnt_ref):
    i = pl.program_id(0)
    packed = jnp.zeros((_H, _W), jnp.float32)
    ones_acc = jnp.zeros((_H, _W), jnp.float32)
    for k in range(_GEN_BLK):
        s_idx = i * _GEN_BLK + k
        base = (s_idx.astype(jnp.uint32) * np.uint32(_SEEDS_PER_SAMPLE)
                + np.uint32(_K1))
        bits = _random_bits(lin_ref[...] + base)
        seed = jnp.where(bits < ts_ref[...], jnp.float32(1.0),
                         jnp.float32(0.0))
        colcnt = jnp.dot(seed, m_ref[...], preferred_element_type=jnp.float32)
        wincnt = jnp.dot(n_ref[...], colcnt, preferred_element_type=jnp.float32)
        keep = jnp.where(wincnt < jnp.float32(0.5), jnp.float32(1.0),
                         jnp.float32(0.0))
        packed = packed + keep * jnp.float32(1 << k)
        ones_acc = ones_acc + keep
    mask_ref[0] = packed.astype(jnp.int32)
    tile_ones = jnp.sum(ones_acc).astype(jnp.int32)
    prev = jnp.where(i == 0, jnp.int32(0), cnt_ref[0, 0])
    cnt_ref[0, 0] = prev + tile_ones


_APPLY_BLK = 48


def _apply_kernel(cnt_ref, x_ref, mask_ref, o_ref):
    scale = jnp.float32(_COUNT_M) / cnt_ref[0, 0].astype(jnp.float32)
    for p in range(_APPLY_BLK // _GEN_BLK):
        packed = mask_ref[p]
        for k in range(_GEN_BLK):
            s = p * _GEN_BLK + k
            bit = packed & jnp.int32(1 << k)
            xs = x_ref[s] * scale
            o_ref[s] = jnp.where(bit != 0, xs, jnp.float32(0.0))


def kernel(x, gamma):
    xr = x.reshape(_D, _H, _W)

    ly = np.minimum(np.arange(_H), _HS - 1).astype(np.uint32)[:, None]
    lx = np.arange(_W, dtype=np.uint32)[None, :]
    lin = jnp.asarray(ly * np.uint32(_WS) + lx)
    thresh = (jnp.ceil(jnp.asarray(gamma, jnp.float32) * jnp.float32(8388608.0))
              .astype(jnp.uint32) << np.uint32(9))
    interior = jnp.asarray(
        ((np.arange(_H) < _HS)[:, None] & (np.arange(_W) < _WS)[None, :]))
    ts = jnp.where(interior, thresh, jnp.uint32(0))
    kk = np.arange(_H)
    m_mat = jnp.asarray(((kk[None, :] - kk[:, None] >= 0)
                         & (kk[None, :] - kk[:, None] <= _BS - 1))
                        .astype(np.float32))
    n_mat = m_mat.T

    mask, cnt = pl.pallas_call(
        _mask_kernel,
        grid=(_D // _GEN_BLK,),
        in_specs=[
            pl.BlockSpec((_H, _W), lambda i: (0, 0)),
            pl.BlockSpec((_H, _W), lambda i: (0, 0)),
            pl.BlockSpec((_H, _W), lambda i: (0, 0)),
            pl.BlockSpec((_H, _W), lambda i: (0, 0)),
        ],
        out_specs=[
            pl.BlockSpec((1, _H, _W), lambda i: (i, 0, 0)),
            pl.BlockSpec(memory_space=pltpu.SMEM),
        ],
        out_shape=[
            jax.ShapeDtypeStruct((_D // _GEN_BLK, _H, _W), jnp.int32),
            jax.ShapeDtypeStruct((1, 1), jnp.int32),
        ],
    )(lin, ts, m_mat, n_mat)

    out = pl.pallas_call(
        _apply_kernel,
        grid=(_D // _APPLY_BLK,),
        in_specs=[
            pl.BlockSpec(memory_space=pltpu.SMEM),
            pl.BlockSpec((_APPLY_BLK, _H, _W), lambda i: (i, 0, 0)),
            pl.BlockSpec((_APPLY_BLK // _GEN_BLK, _H, _W), lambda i: (i, 0, 0)),
        ],
        out_specs=pl.BlockSpec((_APPLY_BLK, _H, _W), lambda i: (i, 0, 0)),
        out_shape=jax.ShapeDtypeStruct((_D, _H, _W), jnp.float32),
    )(cnt, xr, mask)

    return out.reshape(x.shape)


# speedup vs baseline: 1.5360x; 1.0042x over previous
import numpy as np

import jax
import jax.numpy as jnp
from jax.experimental import pallas as pl
from jax.experimental.pallas import tpu as pltpu

_B, _C, _H, _W = 8, 192, 224, 224
_BS = 5
_HS, _WS = _H - (_BS - 1), _W - (_BS - 1)
_D = _B * _C
_COUNT_M = float(_D * _H * _W)
_SEEDS_PER_SAMPLE = _HS * _WS

_ROTS = ((13, 15, 26, 6), (17, 29, 16, 24))


def _threefry_key():
    def tf2x32(k0, k1, x0, x1):
        M = 0xFFFFFFFF
        ks = (k0, k1, 0x1BD11BDA ^ k0 ^ k1)
        x0 = (x0 + ks[0]) & M
        x1 = (x1 + ks[1]) & M
        for g in range(5):
            for r in _ROTS[g % 2]:
                x0 = (x0 + x1) & M
                x1 = ((x1 << r) | (x1 >> (32 - r))) & M
                x1 ^= x0
            x0 = (x0 + ks[(g + 1) % 3]) & M
            x1 = (x1 + ks[(g + 2) % 3] + g + 1) & M
        return x0, x1
    return tf2x32(0, 0, 0, 123)


_K0, _K1 = _threefry_key()
_K2 = 0x1BD11BDA ^ _K0 ^ _K1


def _random_bits(x1):
    ks = (np.uint32(_K0), np.uint32(_K1), np.uint32(_K2))
    x0 = jnp.full(x1.shape, ks[0], jnp.uint32)
    for g in range(5):
        for r in _ROTS[g % 2]:
            x0 = x0 + x1
            x1 = (x1 << np.uint32(r)) | (x1 >> np.uint32(32 - r))
            x1 = x1 ^ x0
        x0 = x0 + ks[(g + 1) % 3]
        x1 = x1 + np.uint32((int(ks[(g + 2) % 3]) + g + 1) & 0xFFFFFFFF)
    return x0 ^ x1


_GEN_BLK = 24


def _mask_kernel(lin_ref, ts_ref, m_ref, n_ref, mask_ref, cnt_ref):
    i = pl.program_id(0)
    packed = jnp.zeros((_H, _W), jnp.float32)
    drop_acc = jnp.zeros((_H, _W), jnp.float32)
    for k in range(_GEN_BLK):
        s_idx = i * _GEN_BLK + k
        base = (s_idx.astype(jnp.uint32) * np.uint32(_SEEDS_PER_SAMPLE)
                + np.uint32(_K1))
        bits = _random_bits(lin_ref[...] + base)
        seed = jnp.where(bits < ts_ref[...], jnp.float32(1.0),
                         jnp.float32(0.0))
        colcnt = jnp.dot(seed, m_ref[...], preferred_element_type=jnp.float32)
        wincnt = jnp.dot(n_ref[...], colcnt, preferred_element_type=jnp.float32)
        dropped = jnp.minimum(wincnt, jnp.float32(1.0))
        packed = packed + dropped * jnp.float32(1 << k)
        drop_acc = drop_acc + dropped
    mask_ref[0] = packed.astype(jnp.int32)
    tile_drop = jnp.sum(drop_acc).astype(jnp.int32)
    prev = jnp.where(i == 0, jnp.int32(0), cnt_ref[0, 0])
    cnt_ref[0, 0] = prev + tile_drop


_APPLY_BLK = 48


def _apply_kernel(cnt_ref, x_ref, mask_ref, o_ref):
    ones = jnp.float32(_COUNT_M) - cnt_ref[0, 0].astype(jnp.float32)
    scale = jnp.float32(_COUNT_M) / ones
    for p in range(_APPLY_BLK // _GEN_BLK):
        packed = mask_ref[p]
        for k in range(_GEN_BLK):
            s = p * _GEN_BLK + k
            bit = packed & jnp.int32(1 << k)
            xs = x_ref[s] * scale
            o_ref[s] = jnp.where(bit == 0, xs, jnp.float32(0.0))


def kernel(x, gamma):
    xr = x.reshape(_D, _H, _W)

    ly = np.minimum(np.arange(_H), _HS - 1).astype(np.uint32)[:, None]
    lx = np.arange(_W, dtype=np.uint32)[None, :]
    lin = jnp.asarray(ly * np.uint32(_WS) + lx)
    thresh = (jnp.ceil(jnp.asarray(gamma, jnp.float32) * jnp.float32(8388608.0))
              .astype(jnp.uint32) << np.uint32(9))
    interior = jnp.asarray(
        ((np.arange(_H) < _HS)[:, None] & (np.arange(_W) < _WS)[None, :]))
    ts = jnp.where(interior, thresh, jnp.uint32(0))
    kk = np.arange(_H)
    m_mat = jnp.asarray(((kk[None, :] - kk[:, None] >= 0)
                         & (kk[None, :] - kk[:, None] <= _BS - 1))
                        .astype(np.float32))
    n_mat = m_mat.T

    mask, cnt = pl.pallas_call(
        _mask_kernel,
        grid=(_D // _GEN_BLK,),
        in_specs=[
            pl.BlockSpec((_H, _W), lambda i: (0, 0)),
            pl.BlockSpec((_H, _W), lambda i: (0, 0)),
            pl.BlockSpec((_H, _W), lambda i: (0, 0)),
            pl.BlockSpec((_H, _W), lambda i: (0, 0)),
        ],
        out_specs=[
            pl.BlockSpec((1, _H, _W), lambda i: (i, 0, 0)),
            pl.BlockSpec(memory_space=pltpu.SMEM),
        ],
        out_shape=[
            jax.ShapeDtypeStruct((_D // _GEN_BLK, _H, _W), jnp.int32),
            jax.ShapeDtypeStruct((1, 1), jnp.int32),
        ],
    )(lin, ts, m_mat, n_mat)

    out = pl.pallas_call(
        _apply_kernel,
        grid=(_D // _APPLY_BLK,),
        in_specs=[
            pl.BlockSpec(memory_space=pltpu.SMEM),
            pl.BlockSpec((_APPLY_BLK, _H, _W), lambda i: (i, 0, 0)),
            pl.BlockSpec((_APPLY_BLK // _GEN_BLK, _H, _W), lambda i: (i, 0, 0)),
        ],
        out_specs=pl.BlockSpec((_APPLY_BLK, _H, _W), lambda i: (i, 0, 0)),
        out_shape=jax.ShapeDtypeStruct((_D, _H, _W), jnp.float32),
    )(cnt, xr, mask)

    return out.reshape(x.shape)
